# baseline (speedup 1.0000x reference)
import functools

import jax
import jax.numpy as jnp
from jax import lax
from jax.experimental import pallas as pl
from jax.experimental.pallas import tpu as pltpu
from jax.experimental.pallas import tpu_sc as plsc

B = 16384
D = 32
H = 256
NROWS = 1000000
PACK = 4
_TCP = 2048
_NSTEP = 123
SL = _NSTEP * _TCP

try:
    _info = plsc.get_sparse_core_info()
    _NC, _NS = _info.num_cores, _info.num_subcores
except Exception:
    _NC, _NS = 2, 16
_NW = _NC * _NS
_BPW = B // _NW


def _pack_body(x0, x1, x2, x3, dst):
    for k, src in enumerate((x0, x1, x2, x3)):
        dst[:, k * D:(k + 1) * D] = jnp.transpose(src[...], (1, 0))


def _pack_table(tT):
    _LAST = NROWS // _TCP
    def mk_spec(k):
        return pl.BlockSpec(
            (D, _TCP), lambda i, k=k: (0, jnp.minimum(_NSTEP * k + i, _LAST)))
    return pl.pallas_call(
        _pack_body,
        grid=(_NSTEP,),
        in_specs=[mk_spec(k) for k in range(PACK)],
        out_specs=pl.BlockSpec((_TCP, PACK * D), lambda i: (i, 0)),
        out_shape=jax.ShapeDtypeStruct((SL, PACK * D), jnp.float32),
        compiler_params=pltpu.CompilerParams(
            dimension_semantics=("arbitrary",),
        ),
    )(tT, tT, tT, tT)


_CH = 256
_NCHUNK = _BPW // _CH


_G = 16


def _sc_gather_body(uidp_hbm, midp_hbm, utp_hbm, mtp_hbm, uep_hbm, mep_hbm,
                    idx_vm, rows_u, rows_m, sem):
    wid = lax.axis_index("s") * _NC + lax.axis_index("c")
    base = wid * _BPW

    pltpu.sync_copy(uidp_hbm.at[pl.ds(base, _BPW)], idx_vm.at[0])
    pltpu.sync_copy(midp_hbm.at[pl.ds(base, _BPW)], idx_vm.at[1])

    for c in range(_NCHUNK):
        handles = []
        for g in range(_CH // _G):
            off = c * _CH + g * _G
            iu = idx_vm[0, pl.ds(off, _G)]
            im = idx_vm[1, pl.ds(off, _G)]
            handles.append(pltpu.async_copy(
                utp_hbm.at[iu], rows_u.at[pl.ds(g * _G, _G)], sem))
            handles.append(pltpu.async_copy(
                mtp_hbm.at[im], rows_m.at[pl.ds(g * _G, _G)], sem))
        for h in handles:
            h.wait()
        pltpu.sync_copy(rows_u, uep_hbm.at[pl.ds(base + c * _CH, _CH)])
        pltpu.sync_copy(rows_m, mep_hbm.at[pl.ds(base + c * _CH, _CH)])


@functools.lru_cache(maxsize=1)
def _sc_gather():
    return pl.kernel(
        _sc_gather_body,
        mesh=plsc.VectorSubcoreMesh(core_axis_name="c", subcore_axis_name="s"),
        out_type=[jax.ShapeDtypeStruct((B, PACK * D), jnp.float32)
                  for _ in range(2)],
        scratch_types=[
            pltpu.VMEM((2, _BPW), jnp.int32),
            pltpu.VMEM((_CH, PACK * D), jnp.float32),
            pltpu.VMEM((_CH, PACK * D), jnp.float32),
            pltpu.SemaphoreType.DMA,
        ],
    )


BLK = 2048


def _onehot_embed(ids_1d, table):
    oh = (ids_1d.reshape(BLK, 1) ==
          lax.broadcasted_iota(jnp.int32, (BLK, 32), 1)).astype(jnp.float32)
    return jnp.dot(oh, table, preferred_element_type=jnp.float32)


def _select_packed(packed, slots_1d):
    lo = slots_1d.reshape(BLK, 1)
    out = jnp.zeros((BLK, D), jnp.float32)
    for k in range(PACK):
        mask = (lo == k).astype(jnp.float32)
        out = out + mask * packed[:, k * D:(k + 1) * D]
    return out


def _mlp_body(eup, emp, uid, mid, gid, aid, oid, nid, gt, at_, ot, nt,
              w1, b1, w2, b2, w3, b3, out):
    eu = _select_packed(eup[...], uid[...])
    em = _select_packed(emp[...], mid[...])
    ge = _onehot_embed(gid[...], gt[...])
    ae = _onehot_embed(aid[...], at_[...])
    oe = _onehot_embed(oid[...], ot[...])
    ne = _onehot_embed(nid[...], nt[...])
    x = jnp.concatenate([eu, ge, ae, oe, em, ne], axis=1)
    h = jnp.dot(x, w1[...], preferred_element_type=jnp.float32) + b1[...]
    h = jnp.maximum(h, 0.0)
    h = jnp.dot(h, w2[...], preferred_element_type=jnp.float32) + b2[...]
    h = jnp.maximum(h, 0.0)
    y = jnp.sum(h * w3[...], axis=1) + b3[0, 0]
    out[...] = y


def _mlp(eup, emp, uid, mid, gid, aid, oid, nid, gt, at_, ot, nt,
         W1, b1, W2, b2, W3, b3):
    pk_spec = pl.BlockSpec((BLK, PACK * D), lambda i: (i, 0))
    id_spec = pl.BlockSpec((BLK,), lambda i: (i,))
    full = lambda shape: pl.BlockSpec(shape, lambda i: tuple(0 for _ in shape))
    return pl.pallas_call(
        _mlp_body,
        grid=(B // BLK,),
        in_specs=[pk_spec, pk_spec] + [id_spec] * 6 + [full((32, D))] * 4 + [
            full((6 * D, H)),
            full((H,)),
            full((H, H)),
            full((H,)),
            full((1, H)),
            full((1, 1)),
        ],
        out_specs=pl.BlockSpec((BLK,), lambda i: (i,)),
        out_shape=jax.ShapeDtypeStruct((B,), jnp.float32),
        compiler_params=pltpu.CompilerParams(
            dimension_semantics=("arbitrary",),
        ),
    )(eup, emp, uid, mid, gid, aid, oid, nid, gt, at_, ot, nt,
      W1, b1, W2, b2, W3, b3)


def _pad32(t):
    return jnp.pad(t, ((0, 32 - t.shape[0]), (0, 0)))


def kernel(user_ids, gender_ids, age_ids, occupation_ids, movie_ids, genre_ids,
           user_table, gender_table, age_table, occupation_table, movie_table,
           genre_table, W1, b1, W2, b2, W3, b3):
    uid = jnp.asarray(user_ids, jnp.int32)
    mid = jnp.asarray(movie_ids, jnp.int32)
    gid = jnp.asarray(gender_ids, jnp.int32)
    aid = jnp.asarray(age_ids, jnp.int32)
    oid = jnp.asarray(occupation_ids, jnp.int32)
    nid = jnp.asarray(genre_ids, jnp.int32)
    utp = _pack_table(user_table.T)
    mtp = _pack_table(movie_table.T)
    eup, emp = _sc_gather()(uid % SL, mid % SL, utp, mtp)
    return _mlp(eup, emp, uid // SL, mid // SL, gid, aid, oid, nid,
                _pad32(gender_table), _pad32(age_table),
                _pad32(occupation_table), _pad32(genre_table),
                W1, b1, W2, b2, W3.reshape(1, H), b3.reshape(1, 1))
